# Initial kernel scaffold; baseline (speedup 1.0000x reference)
import functools

import jax
import jax.numpy as jnp
from jax import lax
from jax.experimental import pallas as pl
from jax.experimental.pallas import tpu as pltpu
from jax.experimental.pallas import tpu_sc as plsc

NC = 2
NS = 16
NW = NC * NS
CHUNK = 1024


def _emb_body(b_per_w, nchunk, ids_hbm, table_hbm, out_hbm, idx_v, rows_v,
              gat_sem):
    wid = lax.axis_index("s") * NC + lax.axis_index("c")
    wbase = wid * b_per_w

    def body(g, carry):
        base = pl.multiple_of(wbase + g * CHUNK, CHUNK)
        pltpu.sync_copy(ids_hbm.at[pl.ds(base, CHUNK)], idx_v)
        pltpu.async_copy(table_hbm.at[idx_v], rows_v, gat_sem).wait()
        pltpu.sync_copy(rows_v, out_hbm.at[pl.ds(base, CHUNK)])
        return carry

    lax.fori_loop(0, nchunk, body, 0, unroll=False)


def kernel(ids, weight):
    batch, hist = ids.shape
    vocab, embed = weight.shape
    b_total = batch * hist
    assert b_total % (NW * CHUNK) == 0
    b_per_w = b_total // NW
    nchunk = b_per_w // CHUNK

    ids_flat = ids.reshape(b_total).astype(jnp.int32)

    mesh = plsc.VectorSubcoreMesh(core_axis_name="c", subcore_axis_name="s")
    emb = pl.kernel(
        functools.partial(_emb_body, b_per_w, nchunk),
        out_type=jax.ShapeDtypeStruct((b_total, embed), jnp.float32),
        mesh=mesh,
        scratch_types=[
            pltpu.VMEM((CHUNK,), jnp.int32),
            pltpu.VMEM((CHUNK, embed), jnp.float32),
            pltpu.SemaphoreType.DMA,
        ],
    )
    out = emb(ids_flat, weight)
    return out.reshape(batch, hist, embed)


# speedup vs baseline: 4.8103x; 4.8103x over previous
import functools

import jax
import jax.numpy as jnp
from jax import lax
from jax.experimental import pallas as pl
from jax.experimental.pallas import tpu as pltpu
from jax.experimental.pallas import tpu_sc as plsc

NC = 2
NS = 16
NW = NC * NS
CHUNK = 1024


def _emb_body(b_per_w, nchunk, ids_hbm, table_hbm, out_hbm, idx_v, rows_v,
              gat_sem):
    wid = lax.axis_index("s") * NC + lax.axis_index("c")
    wbase = wid * b_per_w

    def body(g, carry):
        base = pl.multiple_of(wbase + g * CHUNK, CHUNK)
        pltpu.sync_copy(ids_hbm.at[pl.ds(base, CHUNK)], idx_v)
        pltpu.async_copy(table_hbm.at[idx_v], rows_v, gat_sem).wait()
        pltpu.sync_copy(rows_v, out_hbm.at[pl.ds(base, CHUNK)])
        return carry

    lax.fori_loop(0, nchunk, body, 0, unroll=False)


def kernel(ids, weight):
    batch, hist = ids.shape
    vocab, embed = weight.shape
    b_total = batch * hist
    assert b_total % (NW * CHUNK) == 0
    b_per_w = b_total // NW
    nchunk = b_per_w // CHUNK

    ids_flat = ids.reshape(b_total).astype(jnp.int32)

    mesh = plsc.VectorSubcoreMesh(core_axis_name="c", subcore_axis_name="s")
    emb = pl.kernel(
        functools.partial(_emb_body, b_per_w, nchunk),
        out_type=jax.ShapeDtypeStruct((b_total, embed), jnp.float32),
        mesh=mesh,
        scratch_types=[
            pltpu.VMEM((CHUNK,), jnp.int32),
            pltpu.VMEM((CHUNK, embed), jnp.float32),
            pltpu.SemaphoreType.DMA,
        ],
        compiler_params=pltpu.CompilerParams(use_tc_tiling_on_sc=False),
    )
    out = emb(ids_flat, weight)
    return out.reshape(batch, hist, embed)


# speedup vs baseline: 5.0248x; 1.0446x over previous
import functools

import jax
import jax.numpy as jnp
from jax import lax
from jax.experimental import pallas as pl
from jax.experimental.pallas import tpu as pltpu
from jax.experimental.pallas import tpu_sc as plsc

NC = 2
NS = 16
NW = NC * NS
CHUNK = 1600


def _emb_body(b_per_w, nchunk, ids_hbm, table_hbm, out_hbm,
              idx0, idx1, rows0, rows1,
              isem0, isem1, gsem0, gsem1, osem0, osem1):
    wid = lax.axis_index("s") * NC + lax.axis_index("c")
    wbase = wid * b_per_w

    def idx_copy(g, buf, sem):
        base = pl.multiple_of(wbase + g * CHUNK, CHUNK)
        return pltpu.make_async_copy(ids_hbm.at[pl.ds(base, CHUNK)], buf, sem)

    def out_copy(g, buf, sem):
        base = pl.multiple_of(wbase + g * CHUNK, CHUNK)
        return pltpu.make_async_copy(buf, out_hbm.at[pl.ds(base, CHUNK)], sem)

    def gat_copy(idxbuf, rowbuf, sem):
        return pltpu.make_async_copy(table_hbm.at[idxbuf], rowbuf, sem)

    idx_copy(0, idx0, isem0).start()
    idx_copy(1, idx1, isem1).start()
    idx_copy(0, idx0, isem0).wait()
    gat_copy(idx0, rows0, gsem0).start()
    idx_copy(1, idx1, isem1).wait()
    gat_copy(idx1, rows1, gsem1).start()
    gat_copy(idx0, rows0, gsem0).wait()
    out_copy(0, rows0, osem0).start()
    idx_copy(2, idx0, isem0).start()
    gat_copy(idx1, rows1, gsem1).wait()
    out_copy(1, rows1, osem1).start()
    idx_copy(3, idx1, isem1).start()

    def body(g2, carry):
        g = 2 * g2
        idx_copy(g, idx0, isem0).wait()
        out_copy(g - 2, rows0, osem0).wait()
        gat_copy(idx0, rows0, gsem0).start()
        idx_copy(g + 1, idx1, isem1).wait()
        out_copy(g - 1, rows1, osem1).wait()
        gat_copy(idx1, rows1, gsem1).start()
        gat_copy(idx0, rows0, gsem0).wait()
        out_copy(g, rows0, osem0).start()
        idx_copy(g + 2, idx0, isem0).start()
        gat_copy(idx1, rows1, gsem1).wait()
        out_copy(g + 1, rows1, osem1).start()
        idx_copy(g + 3, idx1, isem1).start()
        return carry

    lax.fori_loop(1, nchunk // 2 - 1, body, 0, unroll=False)

    g = nchunk - 2
    idx_copy(g, idx0, isem0).wait()
    out_copy(g - 2, rows0, osem0).wait()
    gat_copy(idx0, rows0, gsem0).start()
    idx_copy(g + 1, idx1, isem1).wait()
    out_copy(g - 1, rows1, osem1).wait()
    gat_copy(idx1, rows1, gsem1).start()
    gat_copy(idx0, rows0, gsem0).wait()
    out_copy(g, rows0, osem0).start()
    gat_copy(idx1, rows1, gsem1).wait()
    out_copy(g + 1, rows1, osem1).start()
    out_copy(g, rows0, osem0).wait()
    out_copy(g + 1, rows1, osem1).wait()


def kernel(ids, weight):
    batch, hist = ids.shape
    vocab, embed = weight.shape
    b_total = batch * hist
    assert b_total % (NW * CHUNK) == 0
    b_per_w = b_total // NW
    nchunk = b_per_w // CHUNK
    assert nchunk >= 4 and nchunk % 2 == 0

    ids_flat = ids.reshape(b_total).astype(jnp.int32)

    mesh = plsc.VectorSubcoreMesh(core_axis_name="c", subcore_axis_name="s")
    emb = pl.kernel(
        functools.partial(_emb_body, b_per_w, nchunk),
        out_type=jax.ShapeDtypeStruct((b_total, embed), jnp.float32),
        mesh=mesh,
        scratch_types=[
            pltpu.VMEM((CHUNK,), jnp.int32),
            pltpu.VMEM((CHUNK,), jnp.int32),
            pltpu.VMEM((CHUNK, embed), jnp.float32),
            pltpu.VMEM((CHUNK, embed), jnp.float32),
            pltpu.SemaphoreType.DMA,
            pltpu.SemaphoreType.DMA,
            pltpu.SemaphoreType.DMA,
            pltpu.SemaphoreType.DMA,
            pltpu.SemaphoreType.DMA,
            pltpu.SemaphoreType.DMA,
        ],
        compiler_params=pltpu.CompilerParams(use_tc_tiling_on_sc=False),
    )
    out = emb(ids_flat, weight)
    return out.reshape(batch, hist, embed)
